# baseline (speedup 1.0000x reference)
import functools

import jax
import jax.numpy as jnp
from jax import lax
from jax.experimental import pallas as pl
from jax.experimental.pallas import tpu as pltpu
from jax.experimental.pallas import tpu_sc as plsc

TEMP_INV = 1.0 / 0.07
B, L, V, TD, FD = 4096, 50, 100000, 384, 4
FDP = 16
HIST = B * L
NC, NS = 2, 16
NW = NC * NS
CH = 128
HIST_PER_W = HIST // NW
POS_PER_W = B // NW
N_HCHUNK = HIST_PER_W // CH


def _gather_sc(idx_hist, idx_pos, title_table, feat_flat):
    mesh = plsc.VectorSubcoreMesh(core_axis_name="c", subcore_axis_name="s")

    @functools.partial(
        pl.kernel,
        mesh=mesh,
        out_type=(
            jax.ShapeDtypeStruct((HIST, TD), jnp.float32),
            jax.ShapeDtypeStruct((FD, HIST), jnp.float32),
            jax.ShapeDtypeStruct((B, TD), jnp.float32),
            jax.ShapeDtypeStruct((FD, B), jnp.float32),
        ),
        scratch_types=[
            pltpu.VMEM((CH,), jnp.int32),
            pltpu.VMEM((FD, CH), jnp.int32),
            pltpu.VMEM((CH, TD), jnp.float32),
            pltpu.VMEM((FD, CH), jnp.float32),
            pltpu.SemaphoreType.DMA,
            pltpu.SemaphoreType.DMA,
        ],
    )
    def k(ih_hbm, ip_hbm, tt_hbm, ft_hbm, oht, ohf, opt, opf, idx_v, idxf_v,
          rows_v, featc_v, sem_t, sem_f):
        wid = lax.axis_index("s") * NC + lax.axis_index("c")

        def do_chunk(idx_src, base, out_t, out_f):
            pltpu.sync_copy(idx_src.at[pl.ds(base, CH)], idx_v)
            cp_t = pltpu.async_copy(tt_hbm.at[idx_v], rows_v, sem_t)
            for j in range(FD):
                for q in range(CH // 16):
                    s = idx_v[pl.ds(q * 16, 16)]
                    idxf_v[j, pl.ds(q * 16, 16)] = s * FD + j
            cps_f = [
                pltpu.async_copy(ft_hbm.at[idxf_v.at[j]], featc_v.at[j], sem_f)
                for j in range(FD)
            ]
            cp_t.wait()
            for cp in cps_f:
                cp.wait()
            pltpu.sync_copy(rows_v, out_t.at[pl.ds(base, CH)])
            for j in range(FD):
                pltpu.sync_copy(featc_v.at[j], out_f.at[j, pl.ds(base, CH)])

        hbase = wid * HIST_PER_W

        def body(g, carry):
            do_chunk(ih_hbm, hbase + g * CH, oht, ohf)
            return carry

        lax.fori_loop(0, N_HCHUNK, body, 0)
        do_chunk(ip_hbm, wid * POS_PER_W, opt, opf)

    return k(idx_hist, idx_pos, title_table, feat_flat)


def _dot_bf16(a, w):
    return jax.lax.dot(a.astype(jnp.bfloat16), w.astype(jnp.bfloat16),
                       preferred_element_type=jnp.float32)


def _item_tower_block(x, c, W1t, b1, W2, b2, W3, b3):
    h = _dot_bf16(x, W1t) + c + b1
    h = jnp.maximum(h, 0.0)
    h = _dot_bf16(h, W2) + b2
    h = jnp.maximum(h, 0.0)
    e = _dot_bf16(h, W3) + b3
    n = jnp.sqrt(jnp.sum(e * e, axis=-1, keepdims=True))
    return e / jnp.maximum(n, 1e-12)


def _tower_pool_body(g_ref, f_ref, r_ref, m_ref, W1t_ref, W1f_ref, b1_ref,
                     W2_ref, b2_ref, W3_ref, b3_ref, out_ref):
    bb = g_ref.shape[1]
    x = g_ref[...].reshape(L * bb, TD)
    W1f = W1f_ref[...]
    c3 = f_ref[0][:, :, None] * W1f[0][None, None, :]
    for j in range(1, FD):
        c3 = c3 + f_ref[j][:, :, None] * W1f[j][None, None, :]
    e = _item_tower_block(x, c3.reshape(L * bb, 256), W1t_ref[...],
                          b1_ref[...], W2_ref[...], b2_ref[...], W3_ref[...],
                          b3_ref[...])
    e3 = e.reshape(L, bb, 64)
    w = r_ref[...] * m_ref[...]
    wn = w / (jnp.sum(w, axis=0, keepdims=True) + 1e-8)
    out_ref[...] = jnp.sum(e3 * wn[:, :, None], axis=0)


def _pos_tower_body(g_ref, f_ref, W1t_ref, W1f_ref, b1_ref, W2_ref, b2_ref,
                    W3_ref, b3_ref, out_ref):
    W1f = W1f_ref[...]
    c = f_ref[0][:, None] * W1f[0][None, :]
    for j in range(1, FD):
        c = c + f_ref[j][:, None] * W1f[j][None, :]
    out_ref[...] = _item_tower_block(
        g_ref[...], c, W1t_ref[...], b1_ref[...],
        W2_ref[...], b2_ref[...], W3_ref[...], b3_ref[...])


def _final_body(p_ref, pe_ref, U1_ref, ub1_ref, U2_ref, ub2_ref, out_ref):
    hp = jax.lax.Precision.HIGHEST
    h = jnp.maximum(
        jax.lax.dot(p_ref[...], U1_ref[...], precision=hp) + ub1_ref[...], 0.0)
    u = jax.lax.dot(h, U2_ref[...], precision=hp) + ub2_ref[...]
    n = jnp.sqrt(jnp.sum(u * u, axis=-1, keepdims=True))
    u = u / jnp.maximum(n, 1e-12)
    out_ref[...] = jax.lax.dot(u, pe_ref[...], precision=hp) * TEMP_INV


def _full(spec):
    return pl.BlockSpec(spec, lambda i: tuple(0 for _ in spec))


def _tower_pool(g3, f3, rT, mT, W1t, W1f, b1, W2, b2, W3, b3):
    BB = 128
    grid = B // BB
    return pl.pallas_call(
        _tower_pool_body,
        grid=(grid,),
        in_specs=[
            pl.BlockSpec((L, BB, TD), lambda i: (0, i, 0)),
            pl.BlockSpec((FD, L, BB), lambda i: (0, 0, i)),
            pl.BlockSpec((L, BB), lambda i: (0, i)),
            pl.BlockSpec((L, BB), lambda i: (0, i)),
            _full((TD, 256)), _full((FD, 256)), _full((256,)),
            _full((256, 128)), _full((128,)),
            _full((128, 64)), _full((64,)),
        ],
        out_specs=pl.BlockSpec((BB, 64), lambda i: (i, 0)),
        out_shape=jax.ShapeDtypeStruct((B, 64), jnp.float32),
    )(g3, f3, rT, mT, W1t, W1f, b1, W2, b2, W3, b3)


def _pos_tower(gp, fp, W1t, W1f, b1, W2, b2, W3, b3):
    BB = 512
    return pl.pallas_call(
        _pos_tower_body,
        grid=(B // BB,),
        in_specs=[
            pl.BlockSpec((BB, TD), lambda i: (i, 0)),
            pl.BlockSpec((FD, BB), lambda i: (0, i)),
            _full((TD, 256)), _full((FD, 256)), _full((256,)),
            _full((256, 128)), _full((128,)),
            _full((128, 64)), _full((64,)),
        ],
        out_specs=pl.BlockSpec((BB, 64), lambda i: (i, 0)),
        out_shape=jax.ShapeDtypeStruct((B, 64), jnp.float32),
    )(gp, fp, W1t, W1f, b1, W2, b2, W3, b3)


def _final(pooled, pos_emb_t, U1, ub1, U2, ub2):
    BB = 512
    return pl.pallas_call(
        _final_body,
        grid=(B // BB,),
        in_specs=[
            pl.BlockSpec((BB, 64), lambda i: (i, 0)),
            _full((64, B)),
            _full((64, 128)), _full((128,)),
            _full((128, 64)), _full((64,)),
        ],
        out_specs=pl.BlockSpec((BB, B), lambda i: (i, 0)),
        out_shape=jax.ShapeDtypeStruct((B, B), jnp.float32),
    )(pooled, pos_emb_t, U1, ub1, U2, ub2)


def kernel(history_items, history_mask, history_ratings, pos_item, title_table,
           feat_table, W1, b1, W2, b2, W3, b3, U1, ub1, U2, ub2):
    idx_hist = history_items.astype(jnp.int32).T.reshape(-1)
    idx_pos = pos_item.astype(jnp.int32)
    W1t = W1[:TD]
    W1f = W1[TD:]

    ht, hf, pt, pf = _gather_sc(idx_hist, idx_pos, title_table,
                                feat_table.reshape(-1))

    g3 = ht.reshape(L, B, TD)
    f3 = hf.reshape(FD, L, B)
    rT = history_ratings.T
    mT = history_mask.T

    pooled = _tower_pool(g3, f3, rT, mT, W1t, W1f, b1, W2, b2, W3, b3)
    pos_emb = _pos_tower(pt, pf, W1t, W1f, b1, W2, b2, W3, b3)
    return _final(pooled, pos_emb.T, U1, ub1, U2, ub2)


# speedup vs baseline: 1.0876x; 1.0876x over previous
import functools

import jax
import jax.numpy as jnp
from jax import lax
from jax.experimental import pallas as pl
from jax.experimental.pallas import tpu as pltpu
from jax.experimental.pallas import tpu_sc as plsc

TEMP_INV = 1.0 / 0.07
B, L, V, TD, FD = 4096, 50, 100000, 384, 4
FDP = 16
HIST = B * L
NC, NS = 2, 16
NW = NC * NS
CH = 128
HIST_PER_W = HIST // NW
POS_PER_W = B // NW
N_HCHUNK = HIST_PER_W // CH


def _gather_sc(idx_hist, idx_pos, title_table, feat_flat):
    mesh = plsc.VectorSubcoreMesh(core_axis_name="c", subcore_axis_name="s")

    @functools.partial(
        pl.kernel,
        mesh=mesh,
        out_type=(
            jax.ShapeDtypeStruct((HIST, TD), jnp.float32),
            jax.ShapeDtypeStruct((FD, HIST), jnp.float32),
            jax.ShapeDtypeStruct((B, TD), jnp.float32),
            jax.ShapeDtypeStruct((FD, B), jnp.float32),
        ),
        scratch_types=[
            pltpu.VMEM((HIST_PER_W,), jnp.int32),
            pltpu.VMEM((2, FD, CH), jnp.int32),
            pltpu.VMEM((2, CH, TD), jnp.float32),
            pltpu.VMEM((2, FD, CH), jnp.float32),
            pltpu.SemaphoreType.DMA,
            pltpu.SemaphoreType.DMA,
            pltpu.SemaphoreType.DMA,
            pltpu.SemaphoreType.DMA,
        ],
    )
    def k(ih_hbm, ip_hbm, tt_hbm, ft_hbm, oht, ohf, opt, opf, idx_all, idxf_v,
          rows_v, featc_v, gsem0, gsem1, wsem0, wsem1):
        wid = lax.axis_index("s") * NC + lax.axis_index("c")
        hbase = wid * HIST_PER_W
        gsem = (gsem0, gsem1)
        wsem = (wsem0, wsem1)

        pltpu.sync_copy(ih_hbm.at[pl.ds(hbase, HIST_PER_W)], idx_all)

        def comp_fidx(c, b):
            for q in range(CH // 16):
                s4 = idx_all[pl.ds(c * CH + q * 16, 16)] * FD
                for j in range(FD):
                    idxf_v[b, j, pl.ds(q * 16, 16)] = s4 + j

        def fire_gather(c, b):
            pltpu.async_copy(tt_hbm.at[idx_all.at[pl.ds(c * CH, CH)]],
                             rows_v.at[b], gsem[b])
            for j in range(FD):
                pltpu.async_copy(ft_hbm.at[idxf_v.at[b, j]],
                                 featc_v.at[b, j], gsem[b])

        def wait_gather(b):
            pltpu.make_async_copy(tt_hbm.at[pl.ds(0, CH)], rows_v.at[b],
                                  gsem[b]).wait()
            for j in range(FD):
                pltpu.make_async_copy(ft_hbm.at[pl.ds(0, CH)],
                                      featc_v.at[b, j], gsem[b]).wait()

        def fire_wb(c, b):
            base = hbase + c * CH
            pltpu.async_copy(rows_v.at[b], oht.at[pl.ds(base, CH)], wsem[b])
            for j in range(FD):
                pltpu.async_copy(featc_v.at[b, j],
                                 ohf.at[j, pl.ds(base, CH)], wsem[b])

        def wait_wb(b):
            pltpu.make_async_copy(rows_v.at[b], oht.at[pl.ds(0, CH)],
                                  wsem[b]).wait()
            for j in range(FD):
                pltpu.make_async_copy(featc_v.at[b, j],
                                      ohf.at[j, pl.ds(0, CH)], wsem[b]).wait()

        comp_fidx(0, 0)
        fire_gather(0, 0)

        def body(g2, carry):
            for b in (0, 1):
                cg = 2 * g2 + b
                nb = 1 - b

                @pl.when(cg + 1 < N_HCHUNK)
                def _fire_next():
                    comp_fidx(cg + 1, nb)

                    @pl.when(cg >= 1)
                    def _drain_prev_wb():
                        wait_wb(nb)

                    fire_gather(cg + 1, nb)

                wait_gather(b)
                fire_wb(cg, b)
            return carry

        lax.fori_loop(0, N_HCHUNK // 2, body, 0)
        wait_wb(0)
        wait_wb(1)

        pltpu.sync_copy(ip_hbm.at[pl.ds(wid * POS_PER_W, CH)],
                        idx_all.at[pl.ds(0, CH)])
        comp_fidx(0, 0)
        fire_gather(0, 0)
        wait_gather(0)
        pbase = wid * POS_PER_W
        pltpu.sync_copy(rows_v.at[0], opt.at[pl.ds(pbase, CH)])
        for j in range(FD):
            pltpu.sync_copy(featc_v.at[0, j], opf.at[j, pl.ds(pbase, CH)])

    return k(idx_hist, idx_pos, title_table, feat_flat)


def _dot_bf16(a, w):
    return jax.lax.dot(a.astype(jnp.bfloat16), w.astype(jnp.bfloat16),
                       preferred_element_type=jnp.float32)


def _item_tower_block(x, c, W1t, b1, W2, b2, W3, b3):
    h = _dot_bf16(x, W1t) + c + b1
    h = jnp.maximum(h, 0.0)
    h = _dot_bf16(h, W2) + b2
    h = jnp.maximum(h, 0.0)
    e = _dot_bf16(h, W3) + b3
    n = jnp.sqrt(jnp.sum(e * e, axis=-1, keepdims=True))
    return e / jnp.maximum(n, 1e-12)


def _tower_pool_body(g_ref, f_ref, r_ref, m_ref, W1t_ref, W1f_ref, b1_ref,
                     W2_ref, b2_ref, W3_ref, b3_ref, out_ref):
    bb = g_ref.shape[1]
    x = g_ref[...].reshape(L * bb, TD)
    W1f = W1f_ref[...]
    c3 = f_ref[0][:, :, None] * W1f[0][None, None, :]
    for j in range(1, FD):
        c3 = c3 + f_ref[j][:, :, None] * W1f[j][None, None, :]
    e = _item_tower_block(x, c3.reshape(L * bb, 256), W1t_ref[...],
                          b1_ref[...], W2_ref[...], b2_ref[...], W3_ref[...],
                          b3_ref[...])
    e3 = e.reshape(L, bb, 64)
    w = r_ref[...] * m_ref[...]
    wn = w / (jnp.sum(w, axis=0, keepdims=True) + 1e-8)
    out_ref[...] = jnp.sum(e3 * wn[:, :, None], axis=0)


def _pos_tower_body(g_ref, f_ref, W1t_ref, W1f_ref, b1_ref, W2_ref, b2_ref,
                    W3_ref, b3_ref, out_ref):
    W1f = W1f_ref[...]
    c = f_ref[0][:, None] * W1f[0][None, :]
    for j in range(1, FD):
        c = c + f_ref[j][:, None] * W1f[j][None, :]
    out_ref[...] = _item_tower_block(
        g_ref[...], c, W1t_ref[...], b1_ref[...],
        W2_ref[...], b2_ref[...], W3_ref[...], b3_ref[...])


def _final_body(p_ref, pe_ref, U1_ref, ub1_ref, U2_ref, ub2_ref, out_ref):
    hp = jax.lax.Precision.HIGHEST
    h = jnp.maximum(
        jax.lax.dot(p_ref[...], U1_ref[...], precision=hp) + ub1_ref[...], 0.0)
    u = jax.lax.dot(h, U2_ref[...], precision=hp) + ub2_ref[...]
    n = jnp.sqrt(jnp.sum(u * u, axis=-1, keepdims=True))
    u = u / jnp.maximum(n, 1e-12)
    out_ref[...] = jax.lax.dot(u, pe_ref[...], precision=hp) * TEMP_INV


def _full(spec):
    return pl.BlockSpec(spec, lambda i: tuple(0 for _ in spec))


def _tower_pool(g3, f3, rT, mT, W1t, W1f, b1, W2, b2, W3, b3):
    BB = 128
    grid = B // BB
    return pl.pallas_call(
        _tower_pool_body,
        grid=(grid,),
        in_specs=[
            pl.BlockSpec((L, BB, TD), lambda i: (0, i, 0)),
            pl.BlockSpec((FD, L, BB), lambda i: (0, 0, i)),
            pl.BlockSpec((L, BB), lambda i: (0, i)),
            pl.BlockSpec((L, BB), lambda i: (0, i)),
            _full((TD, 256)), _full((FD, 256)), _full((256,)),
            _full((256, 128)), _full((128,)),
            _full((128, 64)), _full((64,)),
        ],
        out_specs=pl.BlockSpec((BB, 64), lambda i: (i, 0)),
        out_shape=jax.ShapeDtypeStruct((B, 64), jnp.float32),
    )(g3, f3, rT, mT, W1t, W1f, b1, W2, b2, W3, b3)


def _pos_tower(gp, fp, W1t, W1f, b1, W2, b2, W3, b3):
    BB = 512
    return pl.pallas_call(
        _pos_tower_body,
        grid=(B // BB,),
        in_specs=[
            pl.BlockSpec((BB, TD), lambda i: (i, 0)),
            pl.BlockSpec((FD, BB), lambda i: (0, i)),
            _full((TD, 256)), _full((FD, 256)), _full((256,)),
            _full((256, 128)), _full((128,)),
            _full((128, 64)), _full((64,)),
        ],
        out_specs=pl.BlockSpec((BB, 64), lambda i: (i, 0)),
        out_shape=jax.ShapeDtypeStruct((B, 64), jnp.float32),
    )(gp, fp, W1t, W1f, b1, W2, b2, W3, b3)


def _final(pooled, pos_emb_t, U1, ub1, U2, ub2):
    BB = 512
    return pl.pallas_call(
        _final_body,
        grid=(B // BB,),
        in_specs=[
            pl.BlockSpec((BB, 64), lambda i: (i, 0)),
            _full((64, B)),
            _full((64, 128)), _full((128,)),
            _full((128, 64)), _full((64,)),
        ],
        out_specs=pl.BlockSpec((BB, B), lambda i: (i, 0)),
        out_shape=jax.ShapeDtypeStruct((B, B), jnp.float32),
    )(pooled, pos_emb_t, U1, ub1, U2, ub2)


def kernel(history_items, history_mask, history_ratings, pos_item, title_table,
           feat_table, W1, b1, W2, b2, W3, b3, U1, ub1, U2, ub2):
    idx_hist = history_items.astype(jnp.int32).T.reshape(-1)
    idx_pos = pos_item.astype(jnp.int32)
    W1t = W1[:TD]
    W1f = W1[TD:]

    ht, hf, pt, pf = _gather_sc(idx_hist, idx_pos, title_table,
                                feat_table.reshape(-1))

    g3 = ht.reshape(L, B, TD)
    f3 = hf.reshape(FD, L, B)
    rT = history_ratings.T
    mT = history_mask.T

    pooled = _tower_pool(g3, f3, rT, mT, W1t, W1f, b1, W2, b2, W3, b3)
    pos_emb = _pos_tower(pt, pf, W1t, W1f, b1, W2, b2, W3, b3)
    return _final(pooled, pos_emb.T, U1, ub1, U2, ub2)


# speedup vs baseline: 1.0935x; 1.0054x over previous
import functools

import jax
import jax.numpy as jnp
from jax import lax
from jax.experimental import pallas as pl
from jax.experimental.pallas import tpu as pltpu
from jax.experimental.pallas import tpu_sc as plsc

TEMP_INV = 1.0 / 0.07
B, L, V, TD, FD = 4096, 50, 100000, 384, 4
FDP = 16
HIST = B * L
NC, NS = 2, 16
NW = NC * NS
CH = 128
HIST_PER_W = HIST // NW
POS_PER_W = B // NW
N_HCHUNK = HIST_PER_W // CH


def _make_gather(nslots, with_pos):
    mesh = plsc.VectorSubcoreMesh(core_axis_name="c", subcore_axis_name="s")
    NR = nslots * B
    per_w = NR // NW
    nch = per_w // CH
    out_type = [
        jax.ShapeDtypeStruct((NR, TD), jnp.float32),
        jax.ShapeDtypeStruct((FD, NR), jnp.float32),
    ]
    if with_pos:
        out_type += [
            jax.ShapeDtypeStruct((B, TD), jnp.float32),
            jax.ShapeDtypeStruct((FD, B), jnp.float32),
        ]

    @functools.partial(
        pl.kernel,
        mesh=mesh,
        out_type=tuple(out_type),
        scratch_types=[
            pltpu.VMEM((per_w,), jnp.int32),
            pltpu.VMEM((2, FD, CH), jnp.int32),
            pltpu.VMEM((2, CH, TD), jnp.float32),
            pltpu.VMEM((2, FD, CH), jnp.float32),
            pltpu.SemaphoreType.DMA,
            pltpu.SemaphoreType.DMA,
            pltpu.SemaphoreType.DMA,
            pltpu.SemaphoreType.DMA,
        ],
    )
    def k(ih_hbm, *rest):
        if with_pos:
            (ip_hbm, tt_hbm, ft_hbm, oht, ohf, opt, opf, idx_all, idxf_v,
             rows_v, featc_v, gsem0, gsem1, wsem0, wsem1) = rest
        else:
            (tt_hbm, ft_hbm, oht, ohf, idx_all, idxf_v,
             rows_v, featc_v, gsem0, gsem1, wsem0, wsem1) = rest
        wid = lax.axis_index("s") * NC + lax.axis_index("c")
        hbase = wid * per_w
        gsem = (gsem0, gsem1)
        wsem = (wsem0, wsem1)

        pltpu.sync_copy(ih_hbm.at[pl.ds(hbase, per_w)], idx_all)

        def comp_fidx(c, b):
            for q in range(CH // 16):
                s4 = idx_all[pl.ds(c * CH + q * 16, 16)] * FD
                for j in range(FD):
                    idxf_v[b, j, pl.ds(q * 16, 16)] = s4 + j

        def fire_gather(c, b):
            pltpu.async_copy(tt_hbm.at[idx_all.at[pl.ds(c * CH, CH)]],
                             rows_v.at[b], gsem[b])
            for j in range(FD):
                pltpu.async_copy(ft_hbm.at[idxf_v.at[b, j]],
                                 featc_v.at[b, j], gsem[b])

        def wait_gather(b):
            pltpu.make_async_copy(tt_hbm.at[pl.ds(0, CH)], rows_v.at[b],
                                  gsem[b]).wait()
            for j in range(FD):
                pltpu.make_async_copy(ft_hbm.at[pl.ds(0, CH)],
                                      featc_v.at[b, j], gsem[b]).wait()

        def fire_wb(c, b):
            base = hbase + c * CH
            pltpu.async_copy(rows_v.at[b], oht.at[pl.ds(base, CH)], wsem[b])
            for j in range(FD):
                pltpu.async_copy(featc_v.at[b, j],
                                 ohf.at[j, pl.ds(base, CH)], wsem[b])

        def wait_wb(b):
            pltpu.make_async_copy(rows_v.at[b], oht.at[pl.ds(0, CH)],
                                  wsem[b]).wait()
            for j in range(FD):
                pltpu.make_async_copy(featc_v.at[b, j],
                                      ohf.at[j, pl.ds(0, CH)], wsem[b]).wait()

        comp_fidx(0, 0)
        fire_gather(0, 0)

        def body(g2, carry):
            for b in (0, 1):
                cg = 2 * g2 + b
                nb = 1 - b

                @pl.when(cg + 1 < nch)
                def _fire_next():
                    comp_fidx(cg + 1, nb)

                    @pl.when(cg >= 1)
                    def _drain_prev_wb():
                        wait_wb(nb)

                    fire_gather(cg + 1, nb)

                wait_gather(b)
                fire_wb(cg, b)
            return carry

        lax.fori_loop(0, nch // 2, body, 0)
        if nch % 2:
            wait_gather(0)
            fire_wb(nch - 1, 0)
        wait_wb(0)
        if nch >= 2:
            wait_wb(1)

        if with_pos:
            pltpu.sync_copy(ip_hbm.at[pl.ds(wid * POS_PER_W, CH)],
                            idx_all.at[pl.ds(0, CH)])
            comp_fidx(0, 0)
            fire_gather(0, 0)
            wait_gather(0)
            pbase = wid * POS_PER_W
            pltpu.sync_copy(rows_v.at[0], opt.at[pl.ds(pbase, CH)])
            for j in range(FD):
                pltpu.sync_copy(featc_v.at[0, j], opf.at[j, pl.ds(pbase, CH)])

    return k


def _dot_bf16(a, w):
    return jax.lax.dot(a.astype(jnp.bfloat16), w.astype(jnp.bfloat16),
                       preferred_element_type=jnp.float32)


def _item_tower_block(x, c, W1t, b1, W2, b2, W3, b3):
    h = _dot_bf16(x, W1t) + c + b1
    h = jnp.maximum(h, 0.0)
    h = _dot_bf16(h, W2) + b2
    h = jnp.maximum(h, 0.0)
    e = _dot_bf16(h, W3) + b3
    n = jnp.sqrt(jnp.sum(e * e, axis=-1, keepdims=True))
    return e / jnp.maximum(n, 1e-12)


def _tower_pool_body(g_ref, f_ref, r_ref, m_ref, W1t_ref, W1f_ref, b1_ref,
                     W2_ref, b2_ref, W3_ref, b3_ref, out_ref, *, lo, nsl):
    bb = g_ref.shape[1]
    x = g_ref[...].reshape(nsl * bb, TD)
    W1f = W1f_ref[...]
    c3 = f_ref[0][:, :, None] * W1f[0][None, None, :]
    for j in range(1, FD):
        c3 = c3 + f_ref[j][:, :, None] * W1f[j][None, None, :]
    e = _item_tower_block(x, c3.reshape(nsl * bb, 256), W1t_ref[...],
                          b1_ref[...], W2_ref[...], b2_ref[...], W3_ref[...],
                          b3_ref[...])
    e3 = e.reshape(nsl, bb, 64)
    w = r_ref[...] * m_ref[...]
    wn = w / (jnp.sum(w, axis=0, keepdims=True) + 1e-8)
    out_ref[...] = jnp.sum(e3 * wn[lo:lo + nsl, :, None], axis=0)


def _pos_tower_body(g_ref, f_ref, W1t_ref, W1f_ref, b1_ref, W2_ref, b2_ref,
                    W3_ref, b3_ref, out_ref):
    W1f = W1f_ref[...]
    c = f_ref[0][:, None] * W1f[0][None, :]
    for j in range(1, FD):
        c = c + f_ref[j][:, None] * W1f[j][None, :]
    out_ref[...] = _item_tower_block(
        g_ref[...], c, W1t_ref[...], b1_ref[...],
        W2_ref[...], b2_ref[...], W3_ref[...], b3_ref[...])


def _final_body(*refs):
    (pe_ref, U1_ref, ub1_ref, U2_ref, ub2_ref, out_ref) = refs[-6:]
    p = refs[0][...]
    for r in refs[1:-6]:
        p = p + r[...]
    hp = jax.lax.Precision.HIGHEST
    h = jnp.maximum(
        jax.lax.dot(p, U1_ref[...], precision=hp) + ub1_ref[...], 0.0)
    u = jax.lax.dot(h, U2_ref[...], precision=hp) + ub2_ref[...]
    n = jnp.sqrt(jnp.sum(u * u, axis=-1, keepdims=True))
    u = u / jnp.maximum(n, 1e-12)
    out_ref[...] = jax.lax.dot(u, pe_ref[...], precision=hp) * TEMP_INV


def _full(spec):
    return pl.BlockSpec(spec, lambda i: tuple(0 for _ in spec))


def _tower_pool(g3, f3, rT, mT, W1t, W1f, b1, W2, b2, W3, b3, *, lo, nsl):
    BB = 128
    grid = B // BB
    return pl.pallas_call(
        functools.partial(_tower_pool_body, lo=lo, nsl=nsl),
        grid=(grid,),
        in_specs=[
            pl.BlockSpec((nsl, BB, TD), lambda i: (0, i, 0)),
            pl.BlockSpec((FD, nsl, BB), lambda i: (0, 0, i)),
            pl.BlockSpec((L, BB), lambda i: (0, i)),
            pl.BlockSpec((L, BB), lambda i: (0, i)),
            _full((TD, 256)), _full((FD, 256)), _full((256,)),
            _full((256, 128)), _full((128,)),
            _full((128, 64)), _full((64,)),
        ],
        out_specs=pl.BlockSpec((BB, 64), lambda i: (i, 0)),
        out_shape=jax.ShapeDtypeStruct((B, 64), jnp.float32),
    )(g3, f3, rT, mT, W1t, W1f, b1, W2, b2, W3, b3)


def _pos_tower(gp, fp, W1t, W1f, b1, W2, b2, W3, b3):
    BB = 512
    return pl.pallas_call(
        _pos_tower_body,
        grid=(B // BB,),
        in_specs=[
            pl.BlockSpec((BB, TD), lambda i: (i, 0)),
            pl.BlockSpec((FD, BB), lambda i: (0, i)),
            _full((TD, 256)), _full((FD, 256)), _full((256,)),
            _full((256, 128)), _full((128,)),
            _full((128, 64)), _full((64,)),
        ],
        out_specs=pl.BlockSpec((BB, 64), lambda i: (i, 0)),
        out_shape=jax.ShapeDtypeStruct((B, 64), jnp.float32),
    )(gp, fp, W1t, W1f, b1, W2, b2, W3, b3)


def _final(pooled_parts, pos_emb_t, U1, ub1, U2, ub2):
    BB = 512
    return pl.pallas_call(
        _final_body,
        grid=(B // BB,),
        in_specs=[pl.BlockSpec((BB, 64), lambda i: (i, 0))
                  for _ in pooled_parts] + [
            _full((64, B)),
            _full((64, 128)), _full((128,)),
            _full((128, 64)), _full((64,)),
        ],
        out_specs=pl.BlockSpec((BB, B), lambda i: (i, 0)),
        out_shape=jax.ShapeDtypeStruct((B, B), jnp.float32),
    )(*pooled_parts, pos_emb_t, U1, ub1, U2, ub2)


SPLITS = (25, 25)


def kernel(history_items, history_mask, history_ratings, pos_item, title_table,
           feat_table, W1, b1, W2, b2, W3, b3, U1, ub1, U2, ub2):
    idxT = history_items.astype(jnp.int32).T
    idx_pos = pos_item.astype(jnp.int32)
    W1t = W1[:TD]
    W1f = W1[TD:]
    ff = feat_table.reshape(-1)
    rT = history_ratings.T
    mT = history_mask.T

    parts = []
    pt = pf = None
    lo = 0
    for si, nsl in enumerate(SPLITS):
        idx_s = idxT[lo:lo + nsl].reshape(-1)
        if si == 0:
            ht, hf, pt, pf = _make_gather(nsl, True)(idx_s, idx_pos,
                                                     title_table, ff)
        else:
            ht, hf = _make_gather(nsl, False)(idx_s, title_table, ff)
        parts.append(_tower_pool(ht.reshape(nsl, B, TD),
                                 hf.reshape(FD, nsl, B), rT, mT, W1t, W1f, b1,
                                 W2, b2, W3, b3, lo=lo, nsl=nsl))
        lo += nsl

    pos_emb = _pos_tower(pt, pf, W1t, W1f, b1, W2, b2, W3, b3)
    return _final(parts, pos_emb.T, U1, ub1, U2, ub2)


# speedup vs baseline: 1.2023x; 1.0994x over previous
import functools

import jax
import jax.numpy as jnp
from jax import lax
from jax.experimental import pallas as pl
from jax.experimental.pallas import tpu as pltpu
from jax.experimental.pallas import tpu_sc as plsc

TEMP_INV = 1.0 / 0.07
B, L, V, TD, FD = 4096, 50, 100000, 384, 4
FDP = 16
HIST = B * L
NC, NS = 2, 16
NW = NC * NS
CH = 128
HIST_PER_W = HIST // NW
POS_PER_W = B // NW
N_HCHUNK = HIST_PER_W // CH


def _make_gather(nslots, with_pos, lo_slot):
    mesh = plsc.VectorSubcoreMesh(core_axis_name="c", subcore_axis_name="s")
    NR = nslots * B
    per_w = NR // NW
    nch = per_w // CH
    lo = lo_slot
    out_type = [
        jax.ShapeDtypeStruct((nslots, B, TD), jnp.float32),
        jax.ShapeDtypeStruct((FD, nslots, B), jnp.float32),
    ]
    if with_pos:
        out_type += [
            jax.ShapeDtypeStruct((B, TD), jnp.float32),
            jax.ShapeDtypeStruct((FD, B), jnp.float32),
        ]

    @functools.partial(
        pl.kernel,
        mesh=mesh,
        out_type=tuple(out_type),
        scratch_types=[
            pltpu.VMEM((per_w,), jnp.int32),
            pltpu.VMEM((per_w,), jnp.int32),
            pltpu.VMEM((2, FD, CH), jnp.int32),
            pltpu.VMEM((2, CH, TD), jnp.float32),
            pltpu.VMEM((2, FD, CH), jnp.float32),
            pltpu.SemaphoreType.DMA,
            pltpu.SemaphoreType.DMA,
            pltpu.SemaphoreType.DMA,
            pltpu.SemaphoreType.DMA,
        ],
    )
    def k(ih_hbm, *rest):
        if with_pos:
            (ip_hbm, tt_hbm, ft_hbm, oht, ohf, opt, opf, pos_buf, idx_all,
             idxf_v, rows_v, featc_v, gsem0, gsem1, wsem0, wsem1) = rest
        else:
            (tt_hbm, ft_hbm, oht, ohf, pos_buf, idx_all,
             idxf_v, rows_v, featc_v, gsem0, gsem1, wsem0, wsem1) = rest
        wid = lax.axis_index("s") * NC + lax.axis_index("c")
        hbase = wid * per_w
        gsem = (gsem0, gsem1)
        wsem = (wsem0, wsem1)

        iota16 = lax.iota(jnp.int32, 16)
        for q in range(per_w // 16):
            p0 = hbase + q * 16
            l_loc = p0 // B
            bvec = (p0 - l_loc * B) + iota16
            pos_buf[pl.ds(q * 16, 16)] = bvec * L + (l_loc + lo)
        for c in range(nch):
            pltpu.async_copy(ih_hbm.at[pos_buf.at[pl.ds(c * CH, CH)]],
                             idx_all.at[pl.ds(c * CH, CH)], gsem0)
        for c in range(nch):
            pltpu.make_async_copy(ih_hbm.at[pl.ds(0, CH)],
                                  idx_all.at[pl.ds(0, CH)], gsem0).wait()

        def comp_fidx(c, b):
            for q in range(CH // 16):
                s4 = idx_all[pl.ds(c * CH + q * 16, 16)] * FD
                for j in range(FD):
                    idxf_v[b, j, pl.ds(q * 16, 16)] = s4 + j

        def fire_gather(c, b):
            pltpu.async_copy(tt_hbm.at[idx_all.at[pl.ds(c * CH, CH)]],
                             rows_v.at[b], gsem[b])
            for j in range(FD):
                pltpu.async_copy(ft_hbm.at[idxf_v.at[b, j]],
                                 featc_v.at[b, j], gsem[b])

        def wait_gather(b):
            pltpu.make_async_copy(tt_hbm.at[pl.ds(0, CH)], rows_v.at[b],
                                  gsem[b]).wait()
            for j in range(FD):
                pltpu.make_async_copy(ft_hbm.at[pl.ds(0, CH)],
                                      featc_v.at[b, j], gsem[b]).wait()

        def fire_wb(c, b):
            base = hbase + c * CH
            l_c = base // B
            b0 = base - l_c * B
            pltpu.async_copy(rows_v.at[b], oht.at[l_c, pl.ds(b0, CH)],
                             wsem[b])
            for j in range(FD):
                pltpu.async_copy(featc_v.at[b, j],
                                 ohf.at[j, l_c, pl.ds(b0, CH)], wsem[b])

        def wait_wb(b):
            pltpu.make_async_copy(rows_v.at[b], oht.at[0, pl.ds(0, CH)],
                                  wsem[b]).wait()
            for j in range(FD):
                pltpu.make_async_copy(featc_v.at[b, j],
                                      ohf.at[j, 0, pl.ds(0, CH)],
                                      wsem[b]).wait()

        comp_fidx(0, 0)
        fire_gather(0, 0)

        def body(g2, carry):
            for b in (0, 1):
                cg = 2 * g2 + b
                nb = 1 - b

                @pl.when(cg + 1 < nch)
                def _fire_next():
                    comp_fidx(cg + 1, nb)

                    @pl.when(cg >= 1)
                    def _drain_prev_wb():
                        wait_wb(nb)

                    fire_gather(cg + 1, nb)

                wait_gather(b)
                fire_wb(cg, b)
            return carry

        lax.fori_loop(0, nch // 2, body, 0)
        if nch % 2:
            wait_gather(0)
            fire_wb(nch - 1, 0)
        wait_wb(0)
        if nch >= 2:
            wait_wb(1)

        if with_pos:
            pltpu.sync_copy(ip_hbm.at[pl.ds(wid * POS_PER_W, CH)],
                            idx_all.at[pl.ds(0, CH)])
            comp_fidx(0, 0)
            fire_gather(0, 0)
            wait_gather(0)
            pbase = wid * POS_PER_W
            pltpu.sync_copy(rows_v.at[0], opt.at[pl.ds(pbase, CH)])
            for j in range(FD):
                pltpu.sync_copy(featc_v.at[0, j], opf.at[j, pl.ds(pbase, CH)])

    return k


def _dot_bf16(a, w):
    return jax.lax.dot(a.astype(jnp.bfloat16), w.astype(jnp.bfloat16),
                       preferred_element_type=jnp.float32)


def _item_tower_block(x, c, W1t, b1, W2, b2, W3, b3):
    h = _dot_bf16(x, W1t) + c + b1
    h = jnp.maximum(h, 0.0)
    h = _dot_bf16(h, W2) + b2
    h = jnp.maximum(h, 0.0)
    e = _dot_bf16(h, W3) + b3
    n = jnp.sqrt(jnp.sum(e * e, axis=-1, keepdims=True))
    return e / jnp.maximum(n, 1e-12)


def _tower_pool_body(g_ref, f_ref, r_ref, m_ref, W1t_ref, W1f_ref, b1_ref,
                     W2_ref, b2_ref, W3_ref, b3_ref, out_ref, *, lo, nsl):
    bb = g_ref.shape[1]
    x = g_ref[...].reshape(nsl * bb, TD)
    W1f = W1f_ref[...]
    c3 = f_ref[0][:, :, None] * W1f[0][None, None, :]
    for j in range(1, FD):
        c3 = c3 + f_ref[j][:, :, None] * W1f[j][None, None, :]
    e = _item_tower_block(x, c3.reshape(nsl * bb, 256), W1t_ref[...],
                          b1_ref[...], W2_ref[...], b2_ref[...], W3_ref[...],
                          b3_ref[...])
    e3 = e.reshape(nsl, bb, 64)
    w = r_ref[...] * m_ref[...]
    wn = w / (jnp.sum(w, axis=1, keepdims=True) + 1e-8)
    wnT = jnp.transpose(wn[:, lo:lo + nsl])
    out_ref[...] = jnp.sum(e3 * wnT[:, :, None], axis=0)


def _pos_tower_body(g_ref, f_ref, W1t_ref, W1f_ref, b1_ref, W2_ref, b2_ref,
                    W3_ref, b3_ref, out_ref):
    W1f = W1f_ref[...]
    c = f_ref[0][:, None] * W1f[0][None, :]
    for j in range(1, FD):
        c = c + f_ref[j][:, None] * W1f[j][None, :]
    out_ref[...] = _item_tower_block(
        g_ref[...], c, W1t_ref[...], b1_ref[...],
        W2_ref[...], b2_ref[...], W3_ref[...], b3_ref[...])


def _final_body(*refs):
    (pe_ref, U1_ref, ub1_ref, U2_ref, ub2_ref, out_ref) = refs[-6:]
    p = refs[0][...]
    for r in refs[1:-6]:
        p = p + r[...]
    hp = jax.lax.Precision.HIGHEST
    h = jnp.maximum(
        jax.lax.dot(p, U1_ref[...], precision=hp) + ub1_ref[...], 0.0)
    u = jax.lax.dot(h, U2_ref[...], precision=hp) + ub2_ref[...]
    n = jnp.sqrt(jnp.sum(u * u, axis=-1, keepdims=True))
    u = u / jnp.maximum(n, 1e-12)
    out_ref[...] = jax.lax.dot(u, pe_ref[...], precision=hp) * TEMP_INV


def _full(spec):
    return pl.BlockSpec(spec, lambda i: tuple(0 for _ in spec))


def _tower_pool(g3, f3, r, m, W1t, W1f, b1, W2, b2, W3, b3, *, lo, nsl):
    BB = 128
    grid = B // BB
    return pl.pallas_call(
        functools.partial(_tower_pool_body, lo=lo, nsl=nsl),
        grid=(grid,),
        in_specs=[
            pl.BlockSpec((nsl, BB, TD), lambda i: (0, i, 0)),
            pl.BlockSpec((FD, nsl, BB), lambda i: (0, 0, i)),
            pl.BlockSpec((BB, L), lambda i: (i, 0)),
            pl.BlockSpec((BB, L), lambda i: (i, 0)),
            _full((TD, 256)), _full((FD, 256)), _full((256,)),
            _full((256, 128)), _full((128,)),
            _full((128, 64)), _full((64,)),
        ],
        out_specs=pl.BlockSpec((BB, 64), lambda i: (i, 0)),
        out_shape=jax.ShapeDtypeStruct((B, 64), jnp.float32),
    )(g3, f3, r, m, W1t, W1f, b1, W2, b2, W3, b3)


def _pos_tower(gp, fp, W1t, W1f, b1, W2, b2, W3, b3):
    BB = 512
    return pl.pallas_call(
        _pos_tower_body,
        grid=(B // BB,),
        in_specs=[
            pl.BlockSpec((BB, TD), lambda i: (i, 0)),
            pl.BlockSpec((FD, BB), lambda i: (0, i)),
            _full((TD, 256)), _full((FD, 256)), _full((256,)),
            _full((256, 128)), _full((128,)),
            _full((128, 64)), _full((64,)),
        ],
        out_specs=pl.BlockSpec((BB, 64), lambda i: (i, 0)),
        out_shape=jax.ShapeDtypeStruct((B, 64), jnp.float32),
    )(gp, fp, W1t, W1f, b1, W2, b2, W3, b3)


def _final(pooled_parts, pos_emb_t, U1, ub1, U2, ub2):
    BB = 512
    return pl.pallas_call(
        _final_body,
        grid=(B // BB,),
        in_specs=[pl.BlockSpec((BB, 64), lambda i: (i, 0))
                  for _ in pooled_parts] + [
            _full((64, B)),
            _full((64, 128)), _full((128,)),
            _full((128, 64)), _full((64,)),
        ],
        out_specs=pl.BlockSpec((BB, B), lambda i: (i, 0)),
        out_shape=jax.ShapeDtypeStruct((B, B), jnp.float32),
    )(*pooled_parts, pos_emb_t, U1, ub1, U2, ub2)


SPLITS = (25, 25)


def kernel(history_items, history_mask, history_ratings, pos_item, title_table,
           feat_table, W1, b1, W2, b2, W3, b3, U1, ub1, U2, ub2):
    ih_flat = history_items.astype(jnp.int32).reshape(-1)
    idx_pos = pos_item.astype(jnp.int32)
    W1t = W1[:TD]
    W1f = W1[TD:]
    ff = feat_table.reshape(-1)

    parts = []
    pt = pf = None
    lo = 0
    for si, nsl in enumerate(SPLITS):
        if si == 0:
            ht, hf, pt, pf = _make_gather(nsl, True, lo)(ih_flat, idx_pos,
                                                         title_table, ff)
        else:
            ht, hf = _make_gather(nsl, False, lo)(ih_flat, title_table, ff)
        parts.append(_tower_pool(ht, hf, history_ratings, history_mask,
                                 W1t, W1f, b1, W2, b2, W3, b3, lo=lo, nsl=nsl))
        lo += nsl

    pos_emb = _pos_tower(pt, pf, W1t, W1f, b1, W2, b2, W3, b3)
    return _final(parts, pos_emb.T, U1, ub1, U2, ub2)
